# baseline (speedup 1.0000x reference)
import functools

import jax
import jax.numpy as jnp
from jax import lax
from jax.experimental import pallas as pl
from jax.experimental.pallas import tpu as pltpu
from jax.experimental.pallas import tpu_sc as plsc

N = 10000
NFEAT = 128
NHID = 128

NMAIN = 9984
NTAIL = N - NMAIN
TAILW = 128

_SC_CORES = 2
_SC_SUBCORES = 16
_NW = _SC_CORES * _SC_SUBCORES
_LANES = 16

_ROWS_PER_CHUNK = 8


def _sc_gather_rows(adj, adjtail, index):
    B = index.shape[0]
    b_per_w = B // _NW
    n_chunks = b_per_w // _ROWS_PER_CHUNK
    mesh = plsc.VectorSubcoreMesh(core_axis_name="c", subcore_axis_name="s")

    @functools.partial(
        pl.kernel,
        mesh=mesh,
        out_type=(
            jax.ShapeDtypeStruct((B, NMAIN), jnp.float32),
            jax.ShapeDtypeStruct((B, TAILW), jnp.float32),
        ),
        scratch_types=[
            pltpu.VMEM((b_per_w,), jnp.int32),
            pltpu.VMEM((_ROWS_PER_CHUNK, NMAIN), jnp.float32),
            pltpu.VMEM((b_per_w, TAILW), jnp.float32),
            pltpu.SemaphoreType.DMA,
            pltpu.SemaphoreType.DMA,
        ],
    )
    def gather_kernel(adj_hbm, tail_hbm, idx_hbm, out_hbm, tails_hbm,
                      idx_v, rows_v, tails_v, sem, tsem):
        wid = lax.axis_index("s") * _SC_CORES + lax.axis_index("c")
        base = wid * b_per_w
        pltpu.sync_copy(idx_hbm.at[pl.ds(base, b_per_w)], idx_v)
        tail_cp = pltpu.async_copy(tail_hbm.at[idx_v], tails_v, tsem)
        for c in range(n_chunks):
            pltpu.async_copy(
                adj_hbm.at[
                    idx_v.at[pl.ds(c * _ROWS_PER_CHUNK, _ROWS_PER_CHUNK)],
                    pl.ds(0, NMAIN),
                ],
                rows_v,
                sem,
            ).wait()
            pltpu.sync_copy(
                rows_v,
                out_hbm.at[pl.ds(base + c * _ROWS_PER_CHUNK, _ROWS_PER_CHUNK)],
            )
        tail_cp.wait()
        pltpu.sync_copy(tails_v, tails_hbm.at[pl.ds(base, b_per_w)])

    return gather_kernel(adj, adjtail, index)


_TM = 200


def _pass1_kernel(x_ref, w1_ref, b1_ref, adj_ref, w2_ref, out_ref, xw1):
    @pl.when(pl.program_id(0) == 0)
    def _():
        xw1[...] = jnp.dot(x_ref[...], w1_ref[...],
                           preferred_element_type=jnp.float32)

    h1 = jnp.dot(adj_ref[...], xw1[...], preferred_element_type=jnp.float32)
    h1 = jnp.maximum(h1 + b1_ref[...], 0.0)
    out_ref[...] = jnp.dot(h1, w2_ref[...], preferred_element_type=jnp.float32)


def _pass1(x, W1, b1, adj, W2):
    grid = (N // _TM,)
    return pl.pallas_call(
        _pass1_kernel,
        grid=grid,
        in_specs=[
            pl.BlockSpec((N, NFEAT), lambda i: (0, 0)),
            pl.BlockSpec((NFEAT, NHID), lambda i: (0, 0)),
            pl.BlockSpec((1, NHID), lambda i: (0, 0)),
            pl.BlockSpec((_TM, N), lambda i: (i, 0)),
            pl.BlockSpec((NHID, NHID), lambda i: (0, 0)),
        ],
        out_specs=pl.BlockSpec((_TM, NHID), lambda i: (i, 0)),
        out_shape=jax.ShapeDtypeStruct((N, NHID), jnp.float32),
        scratch_shapes=[pltpu.VMEM((N, NHID), jnp.float32)],
        compiler_params=pltpu.CompilerParams(
            dimension_semantics=("arbitrary",),
        ),
    )(x, W1, b1, adj, W2)


_TK = 1664


def _pass2_kernel(rows2_ref, hw2_ref, tails_ref, hw2t_ref, s_ref, b2_ref,
                  wl_ref, bl_ref, out_ref, acc):
    j = pl.program_id(0)
    nsteps = pl.num_programs(0)

    @pl.when(j == 0)
    def _():
        acc[...] = jnp.zeros_like(acc)

    acc[...] += jnp.dot(rows2_ref[...], hw2_ref[...],
                        preferred_element_type=jnp.float32)

    @pl.when(j == nsteps - 1)
    def _():
        tail = tails_ref[:, TAILW - NTAIL:]
        h2 = acc[...] + b2_ref[...]
        h2 += jnp.dot(tail, hw2t_ref[...], preferred_element_type=jnp.float32)
        z = jnp.concatenate([h2, s_ref[...]], axis=1)
        logits = lax.dot_general(
            z, wl_ref[...], (((1,), (1,)), ((), ())),
            preferred_element_type=jnp.float32) + bl_ref[...]
        m = jnp.max(logits, axis=1, keepdims=True)
        e = logits - m
        lse = jnp.log(jnp.sum(jnp.exp(e), axis=1, keepdims=True))
        out_ref[...] = e - lse


def _pass2(rows2, tails, hw2, s, b2, Wl, bl):
    B = s.shape[0]
    NS = s.shape[1]
    NCLASS = Wl.shape[0]
    grid = (NMAIN // _TK,)
    return pl.pallas_call(
        _pass2_kernel,
        grid=grid,
        in_specs=[
            pl.BlockSpec((B, _TK), lambda j: (0, j)),
            pl.BlockSpec((_TK, NHID), lambda j: (j, 0)),
            pl.BlockSpec((B, TAILW), lambda j: (0, 0)),
            pl.BlockSpec((NTAIL, NHID), lambda j: (NMAIN // NTAIL, 0)),
            pl.BlockSpec((B, NS), lambda j: (0, 0)),
            pl.BlockSpec((1, NHID), lambda j: (0, 0)),
            pl.BlockSpec((NCLASS, NHID + NS), lambda j: (0, 0)),
            pl.BlockSpec((1, NCLASS), lambda j: (0, 0)),
        ],
        out_specs=pl.BlockSpec((B, NCLASS), lambda j: (0, 0)),
        out_shape=jax.ShapeDtypeStruct((B, NCLASS), jnp.float32),
        scratch_shapes=[pltpu.VMEM((B, NHID), jnp.float32)],
        compiler_params=pltpu.CompilerParams(
            dimension_semantics=("arbitrary",),
        ),
    )(rows2, hw2, tails, hw2, s, b2, Wl, bl)


def kernel(s, x, adj, index, W1, b1, W2, b2, Wl, bl):
    adjtail = lax.slice(adj, (0, N - TAILW), (N, N))
    rows2, tails = _sc_gather_rows(adj, adjtail, index.astype(jnp.int32))
    hw2 = _pass1(x, W1, b1.reshape(1, -1), adj, W2)
    return _pass2(rows2, tails, hw2, s, b2.reshape(1, -1), Wl,
                  bl.reshape(1, -1))


# speedup vs baseline: 1.0859x; 1.0859x over previous
import functools

import jax
import jax.numpy as jnp
from jax import lax
from jax.experimental import pallas as pl
from jax.experimental.pallas import tpu as pltpu
from jax.experimental.pallas import tpu_sc as plsc

N = 10000
NFEAT = 128
NHID = 128

NMAIN = 9984
NTAIL = N - NMAIN
TAILW = 128

_SC_CORES = 2
_SC_SUBCORES = 16
_NW = _SC_CORES * _SC_SUBCORES
_LANES = 16

_ROWS_PER_CHUNK = 8
_HALF = NMAIN // 2


def _sc_gather_rows(adj, adjtail, index):
    B = index.shape[0]
    b_per_w = B // _NW
    n_chunks = b_per_w // _ROWS_PER_CHUNK
    pairs_per_chunk = _ROWS_PER_CHUNK // 2
    chunks_per_write = 2
    pairs_per_write = pairs_per_chunk * chunks_per_write
    mesh = plsc.VectorSubcoreMesh(core_axis_name="c", subcore_axis_name="s")

    @functools.partial(
        pl.kernel,
        mesh=mesh,
        compiler_params=pltpu.CompilerParams(needs_layout_passes=False),
        out_type=(
            jax.ShapeDtypeStruct((B // 2, NMAIN), jnp.int32),
            jax.ShapeDtypeStruct((B, TAILW), jnp.float32),
        ),
        scratch_types=[
            pltpu.VMEM((b_per_w,), jnp.int32),
            pltpu.VMEM((_ROWS_PER_CHUNK, _HALF), jnp.float32),
            pltpu.VMEM((pairs_per_write, NMAIN), jnp.int32),
            pltpu.VMEM((b_per_w, TAILW), jnp.float32),
            pltpu.SemaphoreType.DMA,
            pltpu.SemaphoreType.DMA,
        ],
    )
    def gather_kernel(adj_hbm, tail_hbm, idx_hbm, out_hbm, tails_hbm,
                      idx_v, rows_v, packed_v, tails_v, sem, tsem):
        wid = lax.axis_index("s") * _SC_CORES + lax.axis_index("c")
        base = wid * b_per_w
        obase = wid * (b_per_w // 2)
        pltpu.sync_copy(idx_hbm.at[pl.ds(base, b_per_w)], idx_v)
        tail_cp = pltpu.async_copy(tail_hbm.at[idx_v], tails_v, tsem)
        lane_iota = lax.iota(jnp.int32, _LANES)
        for c in range(n_chunks):
            pbase = (c % chunks_per_write) * pairs_per_chunk
            for half in range(2):
                pltpu.async_copy(
                    adj_hbm.at[
                        idx_v.at[pl.ds(c * _ROWS_PER_CHUNK, _ROWS_PER_CHUNK)],
                        pl.ds(half * _HALF, _HALF),
                    ],
                    rows_v,
                    sem,
                ).wait()
                cbase = half * _HALF

                def pack_body(k, carry):
                    col = k * _LANES + lane_iota
                    for r in range(pairs_per_chunk):
                        a = plsc.load_gather(
                            rows_v, [jnp.full((_LANES,), 2 * r, jnp.int32),
                                     col])
                        b = plsc.load_gather(
                            rows_v, [jnp.full((_LANES,), 2 * r + 1,
                                              jnp.int32), col])
                        p = plsc.pack(a, b,
                                      format=plsc.PackFormat.INTERLEAVED)
                        plsc.store_scatter(
                            packed_v,
                            [jnp.full((_LANES,), pbase + r, jnp.int32),
                             cbase + col],
                            plsc.bitcast(p, jnp.int32))
                    return carry

                lax.fori_loop(0, _HALF // _LANES, pack_body, 0)
            if (c + 1) % chunks_per_write == 0:
                pltpu.sync_copy(
                    packed_v,
                    out_hbm.at[pl.ds(
                        obase + (c // chunks_per_write) * pairs_per_write,
                        pairs_per_write)],
                )
        tail_cp.wait()
        pltpu.sync_copy(tails_v, tails_hbm.at[pl.ds(base, b_per_w)])

    return gather_kernel(adj, adjtail, index)


_TM = 400


def _pass1_kernel(x_ref, w1_ref, b1_ref, adj_ref, w2_ref, out_ref, xw1):
    @pl.when(pl.program_id(0) == 0)
    def _():
        xw1[...] = jnp.dot(x_ref[...], w1_ref[...],
                           preferred_element_type=jnp.float32)

    h1 = jnp.dot(adj_ref[...], xw1[...], preferred_element_type=jnp.float32)
    h1 = jnp.maximum(h1 + b1_ref[...], 0.0)
    out_ref[...] = jnp.dot(h1, w2_ref[...], preferred_element_type=jnp.float32)


def _pass1(x, W1, b1, adj, W2):
    grid = (N // _TM,)
    return pl.pallas_call(
        _pass1_kernel,
        grid=grid,
        in_specs=[
            pl.BlockSpec((N, NFEAT), lambda i: (0, 0)),
            pl.BlockSpec((NFEAT, NHID), lambda i: (0, 0)),
            pl.BlockSpec((1, NHID), lambda i: (0, 0)),
            pl.BlockSpec((_TM, N), lambda i: (i, 0)),
            pl.BlockSpec((NHID, NHID), lambda i: (0, 0)),
        ],
        out_specs=pl.BlockSpec((_TM, NHID), lambda i: (i, 0)),
        out_shape=jax.ShapeDtypeStruct((N, NHID), jnp.float32),
        scratch_shapes=[pltpu.VMEM((N, NHID), jnp.float32)],
        compiler_params=pltpu.CompilerParams(
            dimension_semantics=("arbitrary",),
        ),
    )(x, W1, b1, adj, W2)


_TK = 1664


def _pass2_kernel(rows2_ref, hw2_ref, tails_ref, hw2t_ref, s_ref, b2_ref,
                  wl_ref, bl_ref, out_ref, acc):
    j = pl.program_id(0)
    nsteps = pl.num_programs(0)

    @pl.when(j == 0)
    def _():
        acc[...] = jnp.zeros_like(acc)

    r = pltpu.bitcast(rows2_ref[...], jnp.bfloat16)
    h = hw2_ref[...].astype(jnp.bfloat16)
    acc[...] += jnp.dot(r, h, preferred_element_type=jnp.float32)

    @pl.when(j == nsteps - 1)
    def _():
        tail = tails_ref[:, TAILW - NTAIL:]
        h2 = acc[...] + b2_ref[...]
        h2 += jnp.dot(tail, hw2t_ref[...], preferred_element_type=jnp.float32)
        z = jnp.concatenate([h2, s_ref[...]], axis=1)
        logits = lax.dot_general(
            z, wl_ref[...], (((1,), (1,)), ((), ())),
            preferred_element_type=jnp.float32) + bl_ref[...]
        m = jnp.max(logits, axis=1, keepdims=True)
        e = logits - m
        lse = jnp.log(jnp.sum(jnp.exp(e), axis=1, keepdims=True))
        out_ref[...] = e - lse


def _pass2(rows2, tails, hw2, s, b2, Wl, bl):
    B = s.shape[0]
    NS = s.shape[1]
    NCLASS = Wl.shape[0]
    grid = (NMAIN // _TK,)
    return pl.pallas_call(
        _pass2_kernel,
        grid=grid,
        in_specs=[
            pl.BlockSpec((B // 2, _TK), lambda j: (0, j)),
            pl.BlockSpec((_TK, NHID), lambda j: (j, 0)),
            pl.BlockSpec((B, TAILW), lambda j: (0, 0)),
            pl.BlockSpec((NTAIL, NHID), lambda j: (NMAIN // NTAIL, 0)),
            pl.BlockSpec((B, NS), lambda j: (0, 0)),
            pl.BlockSpec((1, NHID), lambda j: (0, 0)),
            pl.BlockSpec((NCLASS, NHID + NS), lambda j: (0, 0)),
            pl.BlockSpec((1, NCLASS), lambda j: (0, 0)),
        ],
        out_specs=pl.BlockSpec((B, NCLASS), lambda j: (0, 0)),
        out_shape=jax.ShapeDtypeStruct((B, NCLASS), jnp.float32),
        scratch_shapes=[pltpu.VMEM((B, NHID), jnp.float32)],
        compiler_params=pltpu.CompilerParams(
            dimension_semantics=("arbitrary",),
        ),
    )(rows2, hw2, tails, hw2, s, b2, Wl, bl)


def kernel(s, x, adj, index, W1, b1, W2, b2, Wl, bl):
    adjtail = lax.slice(adj, (0, N - TAILW), (N, N))
    rows2, tails = _sc_gather_rows(adj, adjtail, index.astype(jnp.int32))
    hw2 = _pass1(x, W1, b1.reshape(1, -1), adj, W2)
    return _pass2(rows2, tails, hw2, s, b2.reshape(1, -1), Wl,
                  bl.reshape(1, -1))


# speedup vs baseline: 1.0915x; 1.0052x over previous
import functools

import jax
import jax.numpy as jnp
from jax import lax
from jax.experimental import pallas as pl
from jax.experimental.pallas import tpu as pltpu
from jax.experimental.pallas import tpu_sc as plsc

N = 10000
NFEAT = 128
NHID = 128

NMAIN = 9984
NTAIL = N - NMAIN
TAILW = 128

_SC_CORES = 2
_SC_SUBCORES = 16
_NW = _SC_CORES * _SC_SUBCORES
_LANES = 16

_ROWS_PER_CHUNK = 8
_HALF = NMAIN // 2


def _sc_gather_rows(adj, adjtail, index):
    B = index.shape[0]
    b_per_w = B // _NW
    n_chunks = b_per_w // _ROWS_PER_CHUNK
    pairs_per_chunk = _ROWS_PER_CHUNK // 2
    chunks_per_write = 2
    pairs_per_write = pairs_per_chunk * chunks_per_write
    mesh = plsc.VectorSubcoreMesh(core_axis_name="c", subcore_axis_name="s")

    @functools.partial(
        pl.kernel,
        mesh=mesh,
        compiler_params=pltpu.CompilerParams(needs_layout_passes=False),
        out_type=(
            jax.ShapeDtypeStruct((B // 2, NMAIN), jnp.int32),
            jax.ShapeDtypeStruct((B, TAILW), jnp.float32),
        ),
        scratch_types=[
            pltpu.VMEM((b_per_w,), jnp.int32),
            pltpu.VMEM((_ROWS_PER_CHUNK, _HALF), jnp.float32),
            pltpu.VMEM((pairs_per_write, NMAIN), jnp.int32),
            pltpu.VMEM((b_per_w, TAILW), jnp.float32),
            pltpu.SemaphoreType.DMA,
            pltpu.SemaphoreType.DMA,
        ],
    )
    def gather_kernel(adj_hbm, tail_hbm, idx_hbm, out_hbm, tails_hbm,
                      idx_v, rows_v, packed_v, tails_v, sem, tsem):
        wid = lax.axis_index("s") * _SC_CORES + lax.axis_index("c")
        base = wid * b_per_w
        obase = wid * (b_per_w // 2)
        pltpu.sync_copy(idx_hbm.at[pl.ds(base, b_per_w)], idx_v)
        tail_cp = pltpu.async_copy(tail_hbm.at[idx_v], tails_v, tsem)
        lane_iota = lax.iota(jnp.int32, _LANES)
        for c in range(n_chunks):
            pbase = (c % chunks_per_write) * pairs_per_chunk
            for half in range(2):
                pltpu.async_copy(
                    adj_hbm.at[
                        idx_v.at[pl.ds(c * _ROWS_PER_CHUNK, _ROWS_PER_CHUNK)],
                        pl.ds(half * _HALF, _HALF),
                    ],
                    rows_v,
                    sem,
                ).wait()
                cbase = half * _HALF

                def pack_body(k, carry):
                    col = k * _LANES + lane_iota
                    for r in range(pairs_per_chunk):
                        a = plsc.load_gather(
                            rows_v, [jnp.full((_LANES,), 2 * r, jnp.int32),
                                     col])
                        b = plsc.load_gather(
                            rows_v, [jnp.full((_LANES,), 2 * r + 1,
                                              jnp.int32), col])
                        p = plsc.pack(a, b,
                                      format=plsc.PackFormat.INTERLEAVED)
                        plsc.store_scatter(
                            packed_v,
                            [jnp.full((_LANES,), pbase + r, jnp.int32),
                             cbase + col],
                            plsc.bitcast(p, jnp.int32))
                    return carry

                lax.fori_loop(0, _HALF // _LANES, pack_body, 0)
            if (c + 1) % chunks_per_write == 0:
                pltpu.sync_copy(
                    packed_v,
                    out_hbm.at[pl.ds(
                        obase + (c // chunks_per_write) * pairs_per_write,
                        pairs_per_write)],
                )
        tail_cp.wait()
        pltpu.sync_copy(tails_v, tails_hbm.at[pl.ds(base, b_per_w)])

    return gather_kernel(adj, adjtail, index)


_TM = 400


def _pass1_kernel(x_ref, w1_ref, b1_ref, adj_ref, w2_ref, out_ref, xw1):
    @pl.when(pl.program_id(0) == 0)
    def _():
        xw1[...] = jnp.dot(x_ref[...], w1_ref[...],
                           preferred_element_type=jnp.float32)

    h1 = jnp.dot(adj_ref[...], xw1[...], preferred_element_type=jnp.float32)
    h1 = jnp.maximum(h1 + b1_ref[...], 0.0)
    hw2 = jnp.dot(h1, w2_ref[...], preferred_element_type=jnp.float32)
    out_ref[...] = hw2.astype(jnp.bfloat16)


def _pass1(x, W1, b1, adj, W2):
    grid = (N // _TM,)
    return pl.pallas_call(
        _pass1_kernel,
        grid=grid,
        in_specs=[
            pl.BlockSpec((N, NFEAT), lambda i: (0, 0)),
            pl.BlockSpec((NFEAT, NHID), lambda i: (0, 0)),
            pl.BlockSpec((1, NHID), lambda i: (0, 0)),
            pl.BlockSpec((_TM, N), lambda i: (i, 0)),
            pl.BlockSpec((NHID, NHID), lambda i: (0, 0)),
        ],
        out_specs=pl.BlockSpec((_TM, NHID), lambda i: (i, 0)),
        out_shape=jax.ShapeDtypeStruct((N, NHID), jnp.bfloat16),
        scratch_shapes=[pltpu.VMEM((N, NHID), jnp.float32)],
        compiler_params=pltpu.CompilerParams(
            dimension_semantics=("arbitrary",),
        ),
    )(x, W1, b1, adj, W2)


_TK = 1664


def _pass2_kernel(rows2_ref, hw2_ref, tails_ref, hw2t_ref, s_ref, b2_ref,
                  wl_ref, bl_ref, out_ref, acc):
    j = pl.program_id(0)
    nsteps = pl.num_programs(0)

    @pl.when(j == 0)
    def _():
        acc[...] = jnp.zeros_like(acc)

    r = pltpu.bitcast(rows2_ref[...], jnp.bfloat16)
    acc[...] += jnp.dot(r, hw2_ref[...], preferred_element_type=jnp.float32)

    @pl.when(j == nsteps - 1)
    def _():
        tail = tails_ref[:, TAILW - NTAIL:]
        h2 = acc[...] + b2_ref[...]
        h2 += jnp.dot(tail.astype(jnp.bfloat16), hw2t_ref[...],
                      preferred_element_type=jnp.float32)
        z = jnp.concatenate([h2, s_ref[...]], axis=1)
        logits = lax.dot_general(
            z, wl_ref[...], (((1,), (1,)), ((), ())),
            preferred_element_type=jnp.float32) + bl_ref[...]
        m = jnp.max(logits, axis=1, keepdims=True)
        e = logits - m
        lse = jnp.log(jnp.sum(jnp.exp(e), axis=1, keepdims=True))
        out_ref[...] = e - lse


def _pass2(rows2, tails, hw2, s, b2, Wl, bl):
    B = s.shape[0]
    NS = s.shape[1]
    NCLASS = Wl.shape[0]
    grid = (NMAIN // _TK,)
    return pl.pallas_call(
        _pass2_kernel,
        grid=grid,
        in_specs=[
            pl.BlockSpec((B // 2, _TK), lambda j: (0, j)),
            pl.BlockSpec((_TK, NHID), lambda j: (j, 0)),
            pl.BlockSpec((B, TAILW), lambda j: (0, 0)),
            pl.BlockSpec((NTAIL, NHID), lambda j: (NMAIN // NTAIL, 0)),
            pl.BlockSpec((B, NS), lambda j: (0, 0)),
            pl.BlockSpec((1, NHID), lambda j: (0, 0)),
            pl.BlockSpec((NCLASS, NHID + NS), lambda j: (0, 0)),
            pl.BlockSpec((1, NCLASS), lambda j: (0, 0)),
        ],
        out_specs=pl.BlockSpec((B, NCLASS), lambda j: (0, 0)),
        out_shape=jax.ShapeDtypeStruct((B, NCLASS), jnp.float32),
        scratch_shapes=[pltpu.VMEM((B, NHID), jnp.float32)],
        compiler_params=pltpu.CompilerParams(
            dimension_semantics=("arbitrary",),
        ),
    )(rows2, hw2, tails, hw2, s, b2, Wl, bl)


def kernel(s, x, adj, index, W1, b1, W2, b2, Wl, bl):
    adjtail = lax.slice(adj, (0, N - TAILW), (N, N))
    rows2, tails = _sc_gather_rows(adj, adjtail, index.astype(jnp.int32))
    hw2 = _pass1(x, W1, b1.reshape(1, -1), adj, W2)
    return _pass2(rows2, tails, hw2, s, b2.reshape(1, -1), Wl,
                  bl.reshape(1, -1))


# speedup vs baseline: 1.0972x; 1.0052x over previous
import functools

import jax
import jax.numpy as jnp
from jax import lax
from jax.experimental import pallas as pl
from jax.experimental.pallas import tpu as pltpu
from jax.experimental.pallas import tpu_sc as plsc

N = 10000
NFEAT = 128
NHID = 128

NMAIN = 9984
NTAIL = N - NMAIN
TAILW = 128

_SC_CORES = 2
_SC_SUBCORES = 16
_NW = _SC_CORES * _SC_SUBCORES
_LANES = 16

_ROWS_PER_CHUNK = 8
_HALF = NMAIN // 2


def _sc_gather_rows(adj, adjtail, index):
    B = index.shape[0]
    b_per_w = B // _NW
    n_chunks = b_per_w // _ROWS_PER_CHUNK
    pairs_per_chunk = _ROWS_PER_CHUNK // 2
    chunks_per_write = 2
    pairs_per_write = pairs_per_chunk * chunks_per_write
    mesh = plsc.VectorSubcoreMesh(core_axis_name="c", subcore_axis_name="s")

    @functools.partial(
        pl.kernel,
        mesh=mesh,
        compiler_params=pltpu.CompilerParams(needs_layout_passes=False),
        out_type=(
            jax.ShapeDtypeStruct((B // 2, NMAIN), jnp.int32),
            jax.ShapeDtypeStruct((B, TAILW), jnp.float32),
        ),
        scratch_types=[
            pltpu.VMEM((b_per_w,), jnp.int32),
            pltpu.VMEM((_ROWS_PER_CHUNK, _HALF), jnp.float32),
            pltpu.VMEM((pairs_per_write, NMAIN), jnp.int32),
            pltpu.VMEM((b_per_w, TAILW), jnp.float32),
            pltpu.SemaphoreType.DMA,
            pltpu.SemaphoreType.DMA,
        ],
    )
    def gather_kernel(adj_hbm, tail_hbm, idx_hbm, out_hbm, tails_hbm,
                      idx_v, rows_v, packed_v, tails_v, sem, tsem):
        wid = lax.axis_index("s") * _SC_CORES + lax.axis_index("c")
        base = wid * b_per_w
        obase = wid * (b_per_w // 2)
        pltpu.sync_copy(idx_hbm.at[pl.ds(base, b_per_w)], idx_v)
        tail_cp = pltpu.async_copy(tail_hbm.at[idx_v], tails_v, tsem)
        lane_iota = lax.iota(jnp.int32, _LANES)
        for c in range(n_chunks):
            pbase = (c % chunks_per_write) * pairs_per_chunk
            for half in range(2):
                pltpu.async_copy(
                    adj_hbm.at[
                        idx_v.at[pl.ds(c * _ROWS_PER_CHUNK, _ROWS_PER_CHUNK)],
                        pl.ds(half * _HALF, _HALF),
                    ],
                    rows_v,
                    sem,
                ).wait()
                cbase = half * _HALF

                def pack_body(k, carry):
                    col = k * _LANES + lane_iota
                    for r in range(pairs_per_chunk):
                        a = plsc.load_gather(
                            rows_v, [jnp.full((_LANES,), 2 * r, jnp.int32),
                                     col])
                        b = plsc.load_gather(
                            rows_v, [jnp.full((_LANES,), 2 * r + 1,
                                              jnp.int32), col])
                        p = plsc.pack(a, b,
                                      format=plsc.PackFormat.INTERLEAVED)
                        plsc.store_scatter(
                            packed_v,
                            [jnp.full((_LANES,), pbase + r, jnp.int32),
                             cbase + col],
                            plsc.bitcast(p, jnp.int32))
                    return carry

                lax.fori_loop(0, _HALF // _LANES, pack_body, 0)
            if (c + 1) % chunks_per_write == 0:
                pltpu.sync_copy(
                    packed_v,
                    out_hbm.at[pl.ds(
                        obase + (c // chunks_per_write) * pairs_per_write,
                        pairs_per_write)],
                )
        tail_cp.wait()
        pltpu.sync_copy(tails_v, tails_hbm.at[pl.ds(base, b_per_w)])

    return gather_kernel(adj, adjtail, index)


_TM = 400


def _pass1_kernel(x_ref, w1_ref, b1_ref, adj_ref, w2_ref, out_ref, xw1):
    @pl.when(pl.program_id(0) == 0)
    def _():
        xw1[...] = jnp.dot(x_ref[...], w1_ref[...],
                           preferred_element_type=jnp.float32)

    h1 = jnp.dot(adj_ref[...], xw1[...], preferred_element_type=jnp.float32)
    h1 = jnp.maximum(h1 + b1_ref[...], 0.0)
    hw2 = jnp.dot(h1, w2_ref[...], preferred_element_type=jnp.float32)
    out_ref[...] = hw2.astype(jnp.bfloat16)


def _pass1(x, W1, b1, adj, W2):
    grid = (N // _TM,)
    return pl.pallas_call(
        _pass1_kernel,
        grid=grid,
        in_specs=[
            pl.BlockSpec((N, NFEAT), lambda i: (0, 0)),
            pl.BlockSpec((NFEAT, NHID), lambda i: (0, 0)),
            pl.BlockSpec((1, NHID), lambda i: (0, 0)),
            pl.BlockSpec((_TM, N), lambda i: (i, 0)),
            pl.BlockSpec((NHID, NHID), lambda i: (0, 0)),
        ],
        out_specs=pl.BlockSpec((_TM, NHID), lambda i: (i, 0)),
        out_shape=jax.ShapeDtypeStruct((N, NHID), jnp.bfloat16),
        scratch_shapes=[pltpu.VMEM((N, NHID), jnp.float32)],
        compiler_params=pltpu.CompilerParams(
            dimension_semantics=("arbitrary",),
        ),
    )(x, W1, b1, adj, W2)


_TK = 3328


def _pass2_kernel(rows2_ref, hw2_ref, tails_ref, hw2t_ref, s_ref, b2_ref,
                  wl_ref, bl_ref, out_ref, acc):
    j = pl.program_id(0)
    nsteps = pl.num_programs(0)

    @pl.when(j == 0)
    def _():
        acc[...] = jnp.zeros_like(acc)

    r = pltpu.bitcast(rows2_ref[...], jnp.bfloat16)
    acc[...] += jnp.dot(r, hw2_ref[...], preferred_element_type=jnp.float32)

    @pl.when(j == nsteps - 1)
    def _():
        tail = tails_ref[:, TAILW - NTAIL:]
        h2 = acc[...] + b2_ref[...]
        h2 += jnp.dot(tail.astype(jnp.bfloat16), hw2t_ref[...],
                      preferred_element_type=jnp.float32)
        z = jnp.concatenate([h2, s_ref[...]], axis=1)
        logits = lax.dot_general(
            z, wl_ref[...], (((1,), (1,)), ((), ())),
            preferred_element_type=jnp.float32) + bl_ref[...]
        m = jnp.max(logits, axis=1, keepdims=True)
        e = logits - m
        lse = jnp.log(jnp.sum(jnp.exp(e), axis=1, keepdims=True))
        out_ref[...] = e - lse


def _pass2(rows2, tails, hw2, s, b2, Wl, bl):
    B = s.shape[0]
    NS = s.shape[1]
    NCLASS = Wl.shape[0]
    grid = (NMAIN // _TK,)
    return pl.pallas_call(
        _pass2_kernel,
        grid=grid,
        in_specs=[
            pl.BlockSpec((B // 2, _TK), lambda j: (0, j)),
            pl.BlockSpec((_TK, NHID), lambda j: (j, 0)),
            pl.BlockSpec((B, TAILW), lambda j: (0, 0)),
            pl.BlockSpec((NTAIL, NHID), lambda j: (NMAIN // NTAIL, 0)),
            pl.BlockSpec((B, NS), lambda j: (0, 0)),
            pl.BlockSpec((1, NHID), lambda j: (0, 0)),
            pl.BlockSpec((NCLASS, NHID + NS), lambda j: (0, 0)),
            pl.BlockSpec((1, NCLASS), lambda j: (0, 0)),
        ],
        out_specs=pl.BlockSpec((B, NCLASS), lambda j: (0, 0)),
        out_shape=jax.ShapeDtypeStruct((B, NCLASS), jnp.float32),
        scratch_shapes=[pltpu.VMEM((B, NHID), jnp.float32)],
        compiler_params=pltpu.CompilerParams(
            dimension_semantics=("arbitrary",),
        ),
    )(rows2, hw2, tails, hw2, s, b2, Wl, bl)


def kernel(s, x, adj, index, W1, b1, W2, b2, Wl, bl):
    adjtail = lax.slice(adj, (0, N - TAILW), (N, N))
    rows2, tails = _sc_gather_rows(adj, adjtail, index.astype(jnp.int32))
    hw2 = _pass1(x, W1, b1.reshape(1, -1), adj, W2)
    return _pass2(rows2, tails, hw2, s, b2.reshape(1, -1), Wl,
                  bl.reshape(1, -1))
